# baseline (speedup 1.0000x reference)
import jax
import jax.numpy as jnp
from jax import lax
from jax.experimental import pallas as pl
from jax.experimental.pallas import tpu as pltpu
from jax.experimental.pallas import tpu_sc as plsc

P = 500000
D = 64
NB = 3
NC = 5
B = 400
NCHUNK = P // B
NW = 32
JROWS = 5
JB = B // JROWS
OW = B * NC

_BASE_N = NCHUNK // NW
_EXTRA = NCHUNK % NW
_MAXN = _BASE_N + (1 if _EXTRA else 0)


def _pair_basis(rows_u, rows_v, p, wv):
    uv = [rows_u[p, pl.ds(16 * k, 16)] * rows_v[p, pl.ds(16 * k, 16)]
          for k in range(D // 16)]
    bs = []
    for i in range(NB):
        t = uv[0] * wv[i][0]
        for k in range(1, D // 16):
            t = t + uv[k] * wv[i][k]
        bs.append(jnp.sum(t))
    return bs


def _sc_body(uf, vf, uidx, vidx, wb, wsb, out,
             uidx_v, vidx_v, urows, vrows, outb, wb_v, ws_v, sems):
    w = lax.axis_index("s") * 2 + lax.axis_index("c")
    nch = _BASE_N + jnp.where(w < _EXTRA, 1, 0)
    ch0 = w * _BASE_N + jnp.minimum(w, _EXTRA)

    pltpu.sync_copy(wb, wb_v)
    pltpu.sync_copy(wsb, ws_v)

    def start_gathers(ci, s):
        pltpu.sync_copy(uidx.at[pl.ds(ci * B, B)], uidx_v.at[s])
        pltpu.sync_copy(vidx.at[pl.ds(ci * B, B)], vidx_v.at[s])
        for j in range(JROWS):
            pltpu.async_copy(uf.at[uidx_v.at[s, pl.ds(j * JB, JB)]],
                             urows.at[s, pl.ds(j * JB, JB)], sems.at[s])
            pltpu.async_copy(vf.at[vidx_v.at[s, pl.ds(j * JB, JB)]],
                             vrows.at[s, pl.ds(j * JB, JB)], sems.at[s])

    def wait_gathers(s):
        for j in range(JROWS):
            pltpu.make_async_copy(uf.at[uidx_v.at[s, pl.ds(j * JB, JB)]],
                                  urows.at[s, pl.ds(j * JB, JB)],
                                  sems.at[s]).wait()
            pltpu.make_async_copy(vf.at[vidx_v.at[s, pl.ds(j * JB, JB)]],
                                  vrows.at[s, pl.ds(j * JB, JB)],
                                  sems.at[s]).wait()

    def compute(ci, s):
        wv = [[wb_v[pl.ds((i * (D // 16) + k) * 128, 16)]
               for k in range(D // 16)] for i in range(NB)]
        wsd = [ws_v[pl.ds(i * 128, 16)] for i in range(NB)]
        lanes = jnp.arange(16, dtype=jnp.int32)
        lo_half = lanes < 8
        omask = (lanes < NC) | ((lanes >= 8) & (lanes < 8 + NC))
        carry0 = (tuple(tuple(wvi) for wvi in wv), tuple(wsd), lo_half, omask)

        def pair_body(j, carry):
            cwv, cwsd, clo, com = carry
            b0 = _pair_basis(urows.at[s], vrows.at[s], 2 * j, cwv)
            b1 = _pair_basis(urows.at[s], vrows.at[s], 2 * j + 1, cwv)
            ovec = jnp.zeros((16,), jnp.float32)
            for i in range(NB):
                bc = jnp.where(clo, jnp.full((16,), b0[i], jnp.float32),
                               jnp.full((16,), b1[i], jnp.float32))
                ovec = ovec + bc * cwsd[i]
            plsc.store_compressed(outb.at[pl.ds(2 * NC * j, 16)], ovec,
                                  mask=com)
            return carry

        lax.fori_loop(0, B // 2, pair_body, carry0, unroll=False)
        pltpu.sync_copy(outb.at[pl.ds(0, OW)], out.at[pl.ds(ci * OW, OW)])

    start_gathers(ch0, 0)

    def two_chunks(k2, _):
        c0 = ch0 + 2 * k2

        @pl.when(2 * k2 < nch)
        def _slot0():
            wait_gathers(0)

            @pl.when(2 * k2 + 1 < nch)
            def _pre1():
                start_gathers(c0 + 1, 1)

            compute(c0, 0)

        @pl.when(2 * k2 + 1 < nch)
        def _slot1():
            wait_gathers(1)

            @pl.when(2 * k2 + 2 < nch)
            def _pre0():
                start_gathers(c0 + 2, 0)

            compute(c0 + 1, 1)

        return _

    lax.fori_loop(0, (_MAXN + 1) // 2, two_chunks, 0, unroll=False)


def kernel(u_features, v_features, u_indices, v_indices, weights,
           weights_scalars):
    wb = jnp.pad(weights.reshape(NB * (D // 16), 16),
                 ((0, 0), (0, 112))).reshape(-1)
    ws8 = jnp.pad(weights_scalars, ((0, 0), (0, 3)))
    wsd = jnp.concatenate([ws8, ws8], axis=1)
    wsb = jnp.pad(wsd, ((0, 0), (0, 112))).reshape(-1)

    mesh = plsc.VectorSubcoreMesh(core_axis_name="c", subcore_axis_name="s")
    f = pl.kernel(
        _sc_body,
        out_type=jax.ShapeDtypeStruct((P * NC,), jnp.float32),
        mesh=mesh,
        compiler_params=pltpu.CompilerParams(
            needs_layout_passes=False, use_tc_tiling_on_sc=False),
        scratch_types=[
            pltpu.VMEM((2, B), jnp.int32),
            pltpu.VMEM((2, B), jnp.int32),
            pltpu.VMEM((2, B, D), jnp.float32),
            pltpu.VMEM((2, B, D), jnp.float32),
            pltpu.VMEM((OW + 16,), jnp.float32),
            pltpu.VMEM((NB * (D // 16) * 128,), jnp.float32),
            pltpu.VMEM((NB * 128,), jnp.float32),
            pltpu.SemaphoreType.DMA((2,)),
        ],
    )
    res = f(u_features, v_features, u_indices, v_indices, wb, wsb)
    return res.reshape(P, NC)


# speedup vs baseline: 1.1359x; 1.1359x over previous
import jax
import jax.numpy as jnp
from jax import lax
from jax.experimental import pallas as pl
from jax.experimental.pallas import tpu as pltpu
from jax.experimental.pallas import tpu_sc as plsc

P = 500000
D = 64
NB = 3
NC = 5
B = 400
NCHUNK = P // B
NW = 32
JROWS = 5
JB = B // JROWS
OW = 16 * (B // 2)

_BASE_N = NCHUNK // NW
_EXTRA = NCHUNK % NW
_MAXN = _BASE_N + (1 if _EXTRA else 0)


def _pair_basis(rows_u, rows_v, p, wv):
    uv = [rows_u[p, pl.ds(16 * k, 16)] * rows_v[p, pl.ds(16 * k, 16)]
          for k in range(D // 16)]
    bs = []
    for i in range(NB):
        t = uv[0] * wv[i][0]
        for k in range(1, D // 16):
            t = t + uv[k] * wv[i][k]
        bs.append(jnp.sum(t))
    return bs


def _sc_body(uf, vf, uidx, vidx, wb, wsb, out,
             uidx_v, vidx_v, urows, vrows, outb, wb_v, ws_v, sems):
    w = lax.axis_index("s") * 2 + lax.axis_index("c")
    nch = _BASE_N + jnp.where(w < _EXTRA, 1, 0)
    ch0 = w * _BASE_N + jnp.minimum(w, _EXTRA)

    pltpu.sync_copy(wb, wb_v)
    pltpu.sync_copy(wsb, ws_v)

    def start_gathers(ci, s):
        pltpu.sync_copy(uidx.at[pl.ds(ci * B, B)], uidx_v.at[s])
        pltpu.sync_copy(vidx.at[pl.ds(ci * B, B)], vidx_v.at[s])
        for j in range(JROWS):
            pltpu.async_copy(uf.at[uidx_v.at[s, pl.ds(j * JB, JB)]],
                             urows.at[s, pl.ds(j * JB, JB)], sems.at[s])
            pltpu.async_copy(vf.at[vidx_v.at[s, pl.ds(j * JB, JB)]],
                             vrows.at[s, pl.ds(j * JB, JB)], sems.at[s])

    def wait_gathers(s):
        for j in range(JROWS):
            pltpu.make_async_copy(uf.at[uidx_v.at[s, pl.ds(j * JB, JB)]],
                                  urows.at[s, pl.ds(j * JB, JB)],
                                  sems.at[s]).wait()
            pltpu.make_async_copy(vf.at[vidx_v.at[s, pl.ds(j * JB, JB)]],
                                  vrows.at[s, pl.ds(j * JB, JB)],
                                  sems.at[s]).wait()

    def compute(ci, s):
        wv = [[wb_v[pl.ds((i * (D // 16) + k) * 128, 16)]
               for k in range(D // 16)] for i in range(NB)]
        wsd = [ws_v[pl.ds(i * 128, 16)] for i in range(NB)]
        lo_half = jnp.arange(16, dtype=jnp.int32) < 8
        carry0 = (tuple(tuple(wvi) for wvi in wv), tuple(wsd), lo_half)

        def pair_body(j, carry):
            cwv, cwsd, clo = carry
            b0 = _pair_basis(urows.at[s], vrows.at[s], 2 * j, cwv)
            b1 = _pair_basis(urows.at[s], vrows.at[s], 2 * j + 1, cwv)
            ovec = jnp.zeros((16,), jnp.float32)
            for i in range(NB):
                bc = jnp.where(clo, jnp.full((16,), b0[i], jnp.float32),
                               jnp.full((16,), b1[i], jnp.float32))
                ovec = ovec + bc * cwsd[i]
            outb[pl.ds(16 * j, 16)] = ovec
            return carry

        lax.fori_loop(0, B // 2, pair_body, carry0, unroll=False)
        pltpu.sync_copy(outb, out.at[pl.ds(ci * OW, OW)])

    start_gathers(ch0, 0)

    def two_chunks(k2, _):
        c0 = ch0 + 2 * k2

        @pl.when(2 * k2 < nch)
        def _slot0():
            wait_gathers(0)

            @pl.when(2 * k2 + 1 < nch)
            def _pre1():
                start_gathers(c0 + 1, 1)

            compute(c0, 0)

        @pl.when(2 * k2 + 1 < nch)
        def _slot1():
            wait_gathers(1)

            @pl.when(2 * k2 + 2 < nch)
            def _pre0():
                start_gathers(c0 + 2, 0)

            compute(c0 + 1, 1)

        return _

    lax.fori_loop(0, (_MAXN + 1) // 2, two_chunks, 0, unroll=False)


def kernel(u_features, v_features, u_indices, v_indices, weights,
           weights_scalars):
    wb = jnp.pad(weights.reshape(NB * (D // 16), 16),
                 ((0, 0), (0, 112))).reshape(-1)
    ws8 = jnp.pad(weights_scalars, ((0, 0), (0, 3)))
    wsd = jnp.concatenate([ws8, ws8], axis=1)
    wsb = jnp.pad(wsd, ((0, 0), (0, 112))).reshape(-1)

    mesh = plsc.VectorSubcoreMesh(core_axis_name="c", subcore_axis_name="s")
    f = pl.kernel(
        _sc_body,
        out_type=jax.ShapeDtypeStruct((P * 8,), jnp.float32),
        mesh=mesh,
        compiler_params=pltpu.CompilerParams(
            needs_layout_passes=False, use_tc_tiling_on_sc=False),
        scratch_types=[
            pltpu.VMEM((2, B), jnp.int32),
            pltpu.VMEM((2, B), jnp.int32),
            pltpu.VMEM((2, B, D), jnp.float32),
            pltpu.VMEM((2, B, D), jnp.float32),
            pltpu.VMEM((OW,), jnp.float32),
            pltpu.VMEM((NB * (D // 16) * 128,), jnp.float32),
            pltpu.VMEM((NB * 128,), jnp.float32),
            pltpu.SemaphoreType.DMA((2,)),
        ],
    )
    res = f(u_features, v_features, u_indices, v_indices, wb, wsb)
    return res.reshape(P, 8)[:, :NC]
